# baseline (speedup 1.0000x reference)
import functools

import jax
import jax.numpy as jnp
from jax import lax
from jax.experimental import pallas as pl
from jax.experimental.pallas import tpu as pltpu
from jax.experimental.pallas import tpu_sc as plsc

NODE = 128
FEAT = 16
D = NODE * FEAT
ROW_TILE = 1600
TC_ROWS = 3200
N_WORKERS = 32


def _tc_kernel(x_ref, w_ref, o_ref):
    o_ref[...] = jnp.dot(x_ref[...], w_ref[...],
                         preferred_element_type=jnp.float32)


def _tc_part(x2, w, tc_rows):
    return pl.pallas_call(
        _tc_kernel,
        grid=(tc_rows // ROW_TILE,),
        in_specs=[
            pl.BlockSpec((ROW_TILE, D), lambda i: (i, 0)),
            pl.BlockSpec((D, NODE), lambda i: (0, 0)),
        ],
        out_specs=pl.BlockSpec((ROW_TILE, NODE), lambda i: (i, 0)),
        out_shape=jax.ShapeDtypeStruct((tc_rows, NODE), jnp.float32),
    )(x2, w)


def _sc_part(x1, e_flat, tc_rows, sc_rows):
    n_groups = sc_rows // 16
    mesh = plsc.VectorSubcoreMesh(core_axis_name="c", subcore_axis_name="s")

    @functools.partial(
        pl.kernel,
        mesh=mesh,
        compiler_params=pltpu.CompilerParams(needs_layout_passes=False),
        out_type=jax.ShapeDtypeStruct((sc_rows * NODE,), jnp.float32),
        scratch_types=[
            pltpu.VMEM((16 * D,), jnp.float32),
            pltpu.VMEM((16 * D,), jnp.float32),
            pltpu.VMEM((16 * NODE,), jnp.float32),
            pltpu.VMEM((D,), jnp.float32),
            pltpu.SemaphoreType.DMA,
            pltpu.SemaphoreType.DMA,
        ],
    )
    def sck(x_hbm, e_hbm, out_hbm, xb0, xb1, obuf, ebuf, sem0, sem1):
        wid = lax.axis_index("s") * 2 + lax.axis_index("c")
        pltpu.sync_copy(e_hbm, ebuf)
        row_base = lax.iota(jnp.int32, 16) * D
        out_base = lax.iota(jnp.int32, 16) * NODE
        xbufs = (xb0, xb1)
        sems = (sem0, sem1)
        my_groups = (n_groups - wid + N_WORKERS - 1) // N_WORKERS

        def in_copy(i, slot):
            g = wid + i * N_WORKERS
            return pltpu.make_async_copy(
                x_hbm.at[pl.ds((tc_rows + g * 16) * D, 16 * D)],
                xbufs[slot], sems[slot])

        for slot in (0, 1):
            @pl.when(slot < my_groups)
            def _():
                in_copy(slot, slot).start()

        def compute(xbuf):
            lane15 = lax.iota(jnp.int32, 16) == 15

            def n_body(n, c2):
                ev = ebuf[pl.ds(n * FEAT, FEAT)]
                for r in range(16):
                    v = xbuf[pl.ds(r * D + n * FEAT, FEAT)]
                    tot = plsc.cumsum(v * ev)
                    plsc.store_scatter(
                        obuf, [jnp.full((16,), r * NODE + n, jnp.int32)],
                        tot, mask=lane15)
                return c2

            lax.fori_loop(0, NODE, n_body, 0)

        def pair_body(i2, carry):
            for slot in (0, 1):
                i = i2 * 2 + slot

                @pl.when(i < my_groups)
                def _():
                    g = wid + i * N_WORKERS
                    in_copy(i, slot).wait()
                    compute(xbufs[slot])
                    pltpu.sync_copy(
                        obuf, out_hbm.at[pl.ds(g * 16 * NODE, 16 * NODE)])

                    @pl.when(i + 2 < my_groups)
                    def _():
                        in_copy(i + 2, slot).start()
            return carry

        lax.fori_loop(0, (my_groups + 1) // 2, pair_body, 0)

    return sck(x1, e_flat)


def kernel(input_tensor, embedding):
    b, s, h, d = input_tensor.shape
    rows = b * s * h
    x2 = input_tensor.reshape(rows, d)
    e_flat = embedding.reshape(d)
    k = jnp.arange(d)
    w = jnp.zeros((d, NODE), jnp.float32).at[k, k // FEAT].set(e_flat)
    tc_out = _tc_part(x2, w, TC_ROWS)
    sc_out = _sc_part(x2.reshape(rows * d), e_flat, TC_ROWS, rows - TC_ROWS)
    return jnp.concatenate(
        [tc_out, sc_out.reshape(rows - TC_ROWS, NODE)],
        axis=0).reshape(b, s, h, NODE)


# speedup vs baseline: 1.2887x; 1.2887x over previous
import functools

import jax
import jax.numpy as jnp
from jax import lax
from jax.experimental import pallas as pl
from jax.experimental.pallas import tpu as pltpu
from jax.experimental.pallas import tpu_sc as plsc

NODE = 128
FEAT = 16
D = NODE * FEAT
ROW_TILE = 1600
TC_ROWS = 3200
N_WORKERS = 32


def _tc_kernel(x_ref, w_ref, o_ref):
    o_ref[...] = jnp.dot(x_ref[...], w_ref[...],
                         preferred_element_type=jnp.float32)


def _tc_part(x2, w, tc_rows):
    return pl.pallas_call(
        _tc_kernel,
        grid=(tc_rows // ROW_TILE,),
        in_specs=[
            pl.BlockSpec((ROW_TILE, D), lambda i: (i, 0)),
            pl.BlockSpec((D, NODE), lambda i: (0, 0)),
        ],
        out_specs=pl.BlockSpec((ROW_TILE, NODE), lambda i: (i, 0)),
        out_shape=jax.ShapeDtypeStruct((tc_rows, NODE), jnp.float32),
    )(x2, w)


PITCH = 2056


def _sc_part(x2, e_flat, tc_rows, sc_rows):
    n_groups = sc_rows // 16
    mesh = plsc.VectorSubcoreMesh(core_axis_name="c", subcore_axis_name="s")

    @functools.partial(
        pl.kernel,
        mesh=mesh,
        compiler_params=pltpu.CompilerParams(needs_layout_passes=False),
        out_type=jax.ShapeDtypeStruct((sc_rows * NODE,), jnp.float32),
        scratch_types=[
            pltpu.VMEM((16, PITCH), jnp.float32),
            pltpu.VMEM((16, PITCH), jnp.float32),
            pltpu.VMEM((16 * NODE,), jnp.float32),
            pltpu.VMEM((D,), jnp.float32),
            pltpu.SemaphoreType.DMA,
            pltpu.SemaphoreType.DMA,
        ],
    )
    def sck(x_hbm, e_hbm, out_hbm, xb0, xb1, obuf, ebuf, sem0, sem1):
        wid = lax.axis_index("s") * 2 + lax.axis_index("c")
        pltpu.sync_copy(e_hbm, ebuf)
        lanes = lax.iota(jnp.int32, 16)
        out_base = lanes * NODE
        xbufs = (xb0, xb1)
        sems = (sem0, sem1)
        my_groups = (n_groups - wid + N_WORKERS - 1) // N_WORKERS

        def in_copy(i, slot):
            g = wid + i * N_WORKERS
            return pltpu.make_async_copy(
                x_hbm.at[pl.ds(tc_rows + g * 16, 16), :],
                xbufs[slot].at[:, pl.ds(0, D)], sems[slot])

        for slot in (0, 1):
            @pl.when(slot < my_groups)
            def _():
                in_copy(slot, slot).start()

        def compute(xbuf):
            def n_body(n4, c2):
                for j in range(4):
                    n = n4 * 4 + j
                    vs = []
                    for f in range(FEAT):
                        colv = jnp.full((16,), n * FEAT + f, jnp.int32)
                        v = plsc.load_gather(xbuf, [lanes, colv])
                        ev = plsc.load_gather(ebuf, [colv])
                        vs.append(v * ev)
                    a0 = (vs[0] + vs[4]) + (vs[8] + vs[12])
                    a1 = (vs[1] + vs[5]) + (vs[9] + vs[13])
                    a2 = (vs[2] + vs[6]) + (vs[10] + vs[14])
                    a3 = (vs[3] + vs[7]) + (vs[11] + vs[15])
                    plsc.store_scatter(
                        obuf, [out_base + jnp.full((16,), n, jnp.int32)],
                        (a0 + a1) + (a2 + a3))
                return c2

            lax.fori_loop(0, NODE // 4, n_body, 0)

        def pair_body(i2, carry):
            for slot in (0, 1):
                i = i2 * 2 + slot

                @pl.when(i < my_groups)
                def _():
                    g = wid + i * N_WORKERS
                    in_copy(i, slot).wait()
                    compute(xbufs[slot])
                    pltpu.sync_copy(
                        obuf, out_hbm.at[pl.ds(g * 16 * NODE, 16 * NODE)])

                    @pl.when(i + 2 < my_groups)
                    def _():
                        in_copy(i + 2, slot).start()
            return carry

        lax.fori_loop(0, (my_groups + 1) // 2, pair_body, 0)

    return sck(x2, e_flat)


def kernel(input_tensor, embedding):
    b, s, h, d = input_tensor.shape
    rows = b * s * h
    x2 = input_tensor.reshape(rows, d)
    e_flat = embedding.reshape(d)
    k = jnp.arange(d)
    w = jnp.zeros((d, NODE), jnp.float32).at[k, k // FEAT].set(e_flat)
    tc_out = _tc_part(x2, w, TC_ROWS)
    sc_out = _sc_part(x2, e_flat, TC_ROWS, rows - TC_ROWS)
    return jnp.concatenate(
        [tc_out, sc_out.reshape(rows - TC_ROWS, NODE)],
        axis=0).reshape(b, s, h, NODE)


# speedup vs baseline: 10.6474x; 8.2622x over previous
import functools

import jax
import jax.numpy as jnp
from jax import lax
from jax.experimental import pallas as pl
from jax.experimental.pallas import tpu as pltpu
from jax.experimental.pallas import tpu_sc as plsc

NODE = 128
FEAT = 16
D = NODE * FEAT
ROW_TILE = 1600
TC_ROWS = 22976
N_WORKERS = 32


def _tc_kernel(x_ref, w_ref, o_ref):
    o_ref[...] = jnp.dot(x_ref[...], w_ref[...],
                         preferred_element_type=jnp.float32)


def _tc_part(x2, w, tc_rows):
    return pl.pallas_call(
        _tc_kernel,
        grid=(pl.cdiv(tc_rows, ROW_TILE),),
        in_specs=[
            pl.BlockSpec((ROW_TILE, D), lambda i: (i, 0)),
            pl.BlockSpec((D, NODE), lambda i: (0, 0)),
        ],
        out_specs=pl.BlockSpec((ROW_TILE, NODE), lambda i: (i, 0)),
        out_shape=jax.ShapeDtypeStruct((tc_rows, NODE), jnp.float32),
    )(x2, w)


PITCH = 2056


def _sc_part(x2, e_flat, tc_rows, sc_rows):
    n_groups = sc_rows // 16
    mesh = plsc.VectorSubcoreMesh(core_axis_name="c", subcore_axis_name="s")

    @functools.partial(
        pl.kernel,
        mesh=mesh,
        compiler_params=pltpu.CompilerParams(needs_layout_passes=False),
        out_type=jax.ShapeDtypeStruct((sc_rows * NODE,), jnp.float32),
        scratch_types=[
            pltpu.VMEM((16, PITCH), jnp.float32),
            pltpu.VMEM((16, PITCH), jnp.float32),
            pltpu.VMEM((16 * NODE,), jnp.float32),
            pltpu.VMEM((D,), jnp.float32),
            pltpu.SemaphoreType.DMA,
            pltpu.SemaphoreType.DMA,
        ],
    )
    def sck(x_hbm, e_hbm, out_hbm, xb0, xb1, obuf, ebuf, sem0, sem1):
        wid = lax.axis_index("s") * 2 + lax.axis_index("c")
        pltpu.sync_copy(e_hbm, ebuf)
        lanes = lax.iota(jnp.int32, 16)
        out_base = lanes * NODE
        xbufs = (xb0, xb1)
        sems = (sem0, sem1)
        my_groups = (n_groups - wid + N_WORKERS - 1) // N_WORKERS

        def in_copy(i, slot):
            g = wid + i * N_WORKERS
            return pltpu.make_async_copy(
                x_hbm.at[pl.ds(tc_rows + g * 16, 16), :],
                xbufs[slot].at[:, pl.ds(0, D)], sems[slot])

        for slot in (0, 1):
            @pl.when(slot < my_groups)
            def _():
                in_copy(slot, slot).start()

        def compute(xbuf):
            def n_body(n4, c2):
                for j in range(4):
                    n = n4 * 4 + j
                    vs = []
                    for f in range(FEAT):
                        colv = jnp.full((16,), n * FEAT + f, jnp.int32)
                        v = plsc.load_gather(xbuf, [lanes, colv])
                        ev = plsc.load_gather(ebuf, [colv])
                        vs.append(v * ev)
                    a0 = (vs[0] + vs[4]) + (vs[8] + vs[12])
                    a1 = (vs[1] + vs[5]) + (vs[9] + vs[13])
                    a2 = (vs[2] + vs[6]) + (vs[10] + vs[14])
                    a3 = (vs[3] + vs[7]) + (vs[11] + vs[15])
                    plsc.store_scatter(
                        obuf, [out_base + jnp.full((16,), n, jnp.int32)],
                        (a0 + a1) + (a2 + a3))
                return c2

            lax.fori_loop(0, NODE // 4, n_body, 0)

        def pair_body(i2, carry):
            for slot in (0, 1):
                i = i2 * 2 + slot

                @pl.when(i < my_groups)
                def _():
                    g = wid + i * N_WORKERS
                    in_copy(i, slot).wait()
                    compute(xbufs[slot])
                    pltpu.sync_copy(
                        obuf, out_hbm.at[pl.ds(g * 16 * NODE, 16 * NODE)])

                    @pl.when(i + 2 < my_groups)
                    def _():
                        in_copy(i + 2, slot).start()
            return carry

        lax.fori_loop(0, (my_groups + 1) // 2, pair_body, 0)

    return sck(x2, e_flat)


def kernel(input_tensor, embedding):
    b, s, h, d = input_tensor.shape
    rows = b * s * h
    x2 = input_tensor.reshape(rows, d)
    e_flat = embedding.reshape(d)
    k = jnp.arange(d)
    w = jnp.zeros((d, NODE), jnp.float32).at[k, k // FEAT].set(e_flat)
    tc_out = _tc_part(x2, w, TC_ROWS)
    sc_out = _sc_part(x2, e_flat, TC_ROWS, rows - TC_ROWS)
    return jnp.concatenate(
        [tc_out, sc_out.reshape(rows - TC_ROWS, NODE)],
        axis=0).reshape(b, s, h, NODE)


# speedup vs baseline: 10.6889x; 1.0039x over previous
import functools

import jax
import jax.numpy as jnp
from jax import lax
from jax.experimental import pallas as pl
from jax.experimental.pallas import tpu as pltpu
from jax.experimental.pallas import tpu_sc as plsc

NODE = 128
FEAT = 16
D = NODE * FEAT
ROW_TILE = 1600
TC_ROWS = 23488
N_WORKERS = 32


def _tc_kernel(x_ref, w_ref, o_ref):
    o_ref[...] = jnp.dot(x_ref[...], w_ref[...],
                         preferred_element_type=jnp.float32)


def _tc_part(x2, w, tc_rows):
    return pl.pallas_call(
        _tc_kernel,
        grid=(pl.cdiv(tc_rows, ROW_TILE),),
        in_specs=[
            pl.BlockSpec((ROW_TILE, D), lambda i: (i, 0)),
            pl.BlockSpec((D, NODE), lambda i: (0, 0)),
        ],
        out_specs=pl.BlockSpec((ROW_TILE, NODE), lambda i: (i, 0)),
        out_shape=jax.ShapeDtypeStruct((tc_rows, NODE), jnp.float32),
    )(x2, w)


PITCH = 2056


def _sc_part(x2, e_flat, tc_rows, sc_rows):
    n_groups = sc_rows // 16
    mesh = plsc.VectorSubcoreMesh(core_axis_name="c", subcore_axis_name="s")

    @functools.partial(
        pl.kernel,
        mesh=mesh,
        compiler_params=pltpu.CompilerParams(needs_layout_passes=False),
        out_type=jax.ShapeDtypeStruct((sc_rows * NODE,), jnp.float32),
        scratch_types=[
            pltpu.VMEM((16, PITCH), jnp.float32),
            pltpu.VMEM((16, PITCH), jnp.float32),
            pltpu.VMEM((16 * NODE,), jnp.float32),
            pltpu.VMEM((D,), jnp.float32),
            pltpu.SemaphoreType.DMA,
            pltpu.SemaphoreType.DMA,
        ],
    )
    def sck(x_hbm, e_hbm, out_hbm, xb0, xb1, obuf, ebuf, sem0, sem1):
        wid = lax.axis_index("s") * 2 + lax.axis_index("c")
        pltpu.sync_copy(e_hbm, ebuf)
        lanes = lax.iota(jnp.int32, 16)
        out_base = lanes * NODE
        xbufs = (xb0, xb1)
        sems = (sem0, sem1)
        my_groups = (n_groups - wid + N_WORKERS - 1) // N_WORKERS

        def in_copy(i, slot):
            g = wid + i * N_WORKERS
            return pltpu.make_async_copy(
                x_hbm.at[pl.ds(tc_rows + g * 16, 16), :],
                xbufs[slot].at[:, pl.ds(0, D)], sems[slot])

        for slot in (0, 1):
            @pl.when(slot < my_groups)
            def _():
                in_copy(slot, slot).start()

        def compute(xbuf):
            def n_body(n4, c2):
                for j in range(4):
                    n = n4 * 4 + j
                    vs = []
                    for f in range(FEAT):
                        colv = jnp.full((16,), n * FEAT + f, jnp.int32)
                        v = plsc.load_gather(xbuf, [lanes, colv])
                        ev = plsc.load_gather(ebuf, [colv])
                        vs.append(v * ev)
                    a0 = (vs[0] + vs[4]) + (vs[8] + vs[12])
                    a1 = (vs[1] + vs[5]) + (vs[9] + vs[13])
                    a2 = (vs[2] + vs[6]) + (vs[10] + vs[14])
                    a3 = (vs[3] + vs[7]) + (vs[11] + vs[15])
                    plsc.store_scatter(
                        obuf, [out_base + jnp.full((16,), n, jnp.int32)],
                        (a0 + a1) + (a2 + a3))
                return c2

            lax.fori_loop(0, NODE // 4, n_body, 0)

        def pair_body(i2, carry):
            for slot in (0, 1):
                i = i2 * 2 + slot

                @pl.when(i < my_groups)
                def _():
                    g = wid + i * N_WORKERS
                    in_copy(i, slot).wait()
                    compute(xbufs[slot])
                    pltpu.sync_copy(
                        obuf, out_hbm.at[pl.ds(g * 16 * NODE, 16 * NODE)])

                    @pl.when(i + 2 < my_groups)
                    def _():
                        in_copy(i + 2, slot).start()
            return carry

        lax.fori_loop(0, (my_groups + 1) // 2, pair_body, 0)

    return sck(x2, e_flat)


def kernel(input_tensor, embedding):
    b, s, h, d = input_tensor.shape
    rows = b * s * h
    x2 = input_tensor.reshape(rows, d)
    e_flat = embedding.reshape(d)
    k = jnp.arange(d)
    w = jnp.zeros((d, NODE), jnp.float32).at[k, k // FEAT].set(e_flat)
    tc_out = _tc_part(x2, w, TC_ROWS)
    sc_out = _sc_part(x2, e_flat, TC_ROWS, rows - TC_ROWS)
    return jnp.concatenate(
        [tc_out, sc_out.reshape(rows - TC_ROWS, NODE)],
        axis=0).reshape(b, s, h, NODE)


# speedup vs baseline: 14.2531x; 1.3335x over previous
import jax
import jax.numpy as jnp
from jax.experimental import pallas as pl

NODE = 128
FEAT = 16
ROW_TILE = 1600


def _tc_kernel(x_ref, w_ref, o_ref):
    o_ref[...] = jnp.dot(x_ref[...], w_ref[...],
                         preferred_element_type=jnp.float32)


def kernel(input_tensor, embedding):
    b, s, h, d = input_tensor.shape
    rows = b * s * h
    x2 = input_tensor.reshape(rows, d)
    k = jnp.arange(d)
    w = jnp.zeros((d, NODE), jnp.float32).at[k, k // FEAT].set(
        embedding.reshape(d))
    grid = rows // ROW_TILE
    out = pl.pallas_call(
        _tc_kernel,
        grid=(grid,),
        in_specs=[
            pl.BlockSpec((ROW_TILE, d), lambda i: (i, 0)),
            pl.BlockSpec((d, NODE), lambda i: (0, 0)),
        ],
        out_specs=pl.BlockSpec((ROW_TILE, NODE), lambda i: (i, 0)),
        out_shape=jax.ShapeDtypeStruct((rows, NODE), jnp.float32),
    )(x2, w)
    return out.reshape(b, s, h, NODE)
